# Initial kernel scaffold; baseline (speedup 1.0000x reference)
import jax
import jax.numpy as jnp
from jax.experimental import pallas as pl


def kernel(node_embedding, segment_ids, Wg, bg, Wt, bt):
    raise NotImplementedError("write your pallas kernel here")


# speedup vs baseline: 3.1355x; 3.1355x over previous
import jax
import jax.numpy as jnp
from jax import lax
from jax.experimental import pallas as pl
from jax.experimental.pallas import tpu as pltpu
from jax.experimental.pallas import tpu_sc as plsc

N = 100000
H = 256
G = 512
S = 512

NC = 2
NS = 16
NW = NC * NS
L = 16

CH = 80
HB = H // L
W = H + L
SEGW = S // NW


def _sc_body(x_hbm, seg_hbm, bounds_hbm, wg_hbm, out_hbm, xin, idxp, bnd, wgv, acc):
    cid = lax.axis_index("c")
    sid = lax.axis_index("s")
    wid = sid * NC + cid

    pltpu.sync_copy(wg_hbm, wgv)
    pltpu.sync_copy(bounds_hbm, bnd)
    wgs = [wgv[pl.ds(L * h, L)] for h in range(HB)]
    bg_init = wgv[pl.ds(H, L)]
    lo = bnd[pl.ds(wid, L)][0]
    hi = bnd[pl.ds(wid + 1, L)][0]

    zero = jnp.zeros((L,), jnp.float32)
    for r in range(SEGW):
        for h in range(W // L):
            acc[r, pl.ds(L * h, L)] = zero

    seg_base = wid * SEGW

    def row_body(r, carry):
        b, dk = carry
        gi = b + r
        seg = idxp[pl.ds((b - ((b // L) * L)) + r, L)][0]
        ls = jnp.clip(seg - seg_base, 0, SEGW - 1)
        xrow = [xin[r, pl.ds(L * h, L)] for h in range(HB)]
        accs = [xrow[j] * wgs[j] for j in range(4)]
        for h in range(4, HB):
            j = h % 4
            accs[j] = accs[j] + xrow[h] * wgs[h]
        av = ((accs[0] + accs[1]) + (accs[2] + accs[3])) + bg_init
        z = jnp.sum(av)
        valid = jnp.logical_and(
            jnp.logical_and(gi >= dk, gi >= lo), gi < hi
        )
        zv = jnp.full((L,), z, jnp.float32)
        gv = (1.0 / (1.0 + jnp.exp(-zv))) * valid.astype(jnp.float32)
        for h in range(HB):
            acc[ls, pl.ds(L * h, L)] = acc[ls, pl.ds(L * h, L)] + xrow[h] * gv
        acc[ls, pl.ds(H, L)] = acc[ls, pl.ds(H, L)] + gv
        return carry

    lo8 = (lo // 8) * 8

    def chunk_body(k, c):
        dk = lo8 + k * CH
        b = pl.multiple_of(jnp.minimum(dk, N - CH), 8)
        b16 = pl.multiple_of((b // L) * L, L)
        pltpu.sync_copy(seg_hbm.at[pl.ds(b16, CH + L)], idxp.at[pl.ds(0, CH + L)])
        pltpu.sync_copy(x_hbm.at[pl.ds(b, CH), :], xin)
        lax.fori_loop(0, CH, row_body, (b, dk))
        return c

    nk = (hi - lo8 + CH - 1) // CH
    lax.fori_loop(0, nk, chunk_body, 0)

    pltpu.sync_copy(acc, out_hbm.at[pl.ds(seg_base, SEGW), :])


def _sc_segment_accumulate(x, seg, bounds, wgext):
    mesh = plsc.VectorSubcoreMesh(
        core_axis_name="c", subcore_axis_name="s", num_cores=NC, num_subcores=NS
    )
    fn = pl.kernel(
        _sc_body,
        out_type=jax.ShapeDtypeStruct((S, W), jnp.float32),
        mesh=mesh,
        scratch_types=[
            pltpu.VMEM((CH, H), jnp.float32),
            pltpu.VMEM((CH + 2 * L,), jnp.int32),
            pltpu.VMEM((NW + L,), jnp.int32),
            pltpu.VMEM((H + L,), jnp.float32),
            pltpu.VMEM((SEGW, W), jnp.float32),
        ],
        compiler_params=pltpu.CompilerParams(needs_layout_passes=False),
    )
    return fn(x, seg, bounds, wgext)


def _tc_body(p_ref, wt_ref, bt_ref, o_ref):
    y = p_ref[:, :H]
    g = p_ref[:, H:H + 1]
    o_ref[...] = (
        jnp.dot(y, wt_ref[...], preferred_element_type=jnp.float32) + g * bt_ref[...]
    )


def kernel(node_embedding, segment_ids, Wg, bg, Wt, bt):
    seg = segment_ids.astype(jnp.int32)
    bounds = jnp.searchsorted(
        seg, jnp.arange(0, S + SEGW, SEGW, dtype=jnp.int32)
    ).astype(jnp.int32)
    bounds = jnp.pad(bounds, (0, NW + L - bounds.shape[0]))
    wgext = jnp.concatenate(
        [Wg[:, 0], jnp.full((L,), bg[0] / L, dtype=jnp.float32)]
    )
    partial = _sc_segment_accumulate(node_embedding, seg, bounds, wgext)
    out = pl.pallas_call(
        _tc_body,
        out_shape=jax.ShapeDtypeStruct((S, G), jnp.float32),
    )(partial, Wt, bt.reshape(1, G))
    return out


# speedup vs baseline: 5.2751x; 1.6824x over previous
import jax
import jax.numpy as jnp
from jax import lax
from jax.experimental import pallas as pl
from jax.experimental.pallas import tpu as pltpu
from jax.experimental.pallas import tpu_sc as plsc

N = 100000
H = 256
G = 512
S = 512

NC = 2
NS = 16
NW = NC * NS
L = 16

CH = 80
RU = 2
HB = H // L
W = H + L
SEGW = S // NW
ZV17 = HB + 1


def _sc_body(
    x_hbm, seg_hbm, bounds_hbm, wg_hbm, out_hbm,
    xin, idxp, bnd, wgv, acc, semx, semi,
):
    cid = lax.axis_index("c")
    sid = lax.axis_index("s")
    wid = sid * NC + cid

    pltpu.sync_copy(wg_hbm, wgv)
    pltpu.sync_copy(bounds_hbm, bnd)
    wgs = [wgv[pl.ds(L * h, L)] for h in range(HB)]
    bg_init = wgv[pl.ds(H, L)]
    lo = bnd[pl.ds(wid, L)][0]
    hi = bnd[pl.ds(wid + 1, L)][0]

    zero = jnp.zeros((L,), jnp.float32)
    for r in range(SEGW):
        for h in range(W // L):
            acc[r, pl.ds(L * h, L)] = zero

    seg_base = wid * SEGW

    lo8 = (lo // 8) * 8
    nk = (hi - lo8 + CH - 1) // CH

    def bases(k):
        dk = lo8 + k * CH
        b = pl.multiple_of(jnp.minimum(dk, N - CH), 8)
        b16 = pl.multiple_of((b // L) * L, L)
        return dk, b, b16

    def dma_start(k, kb):
        _, b, b16 = bases(k)
        pltpu.async_copy(
            seg_hbm.at[pl.ds(b16, CH + L)],
            idxp.at[kb, pl.ds(0, CH + L)],
            semi.at[kb],
        )
        pltpu.async_copy(x_hbm.at[pl.ds(b, CH), :], xin.at[kb], semx.at[kb])

    def dma_wait(k, kb):
        _, b, b16 = bases(k)
        pltpu.make_async_copy(
            seg_hbm.at[pl.ds(b16, CH + L)],
            idxp.at[kb, pl.ds(0, CH + L)],
            semi.at[kb],
        ).wait()
        pltpu.make_async_copy(
            x_hbm.at[pl.ds(b, CH), :], xin.at[kb], semx.at[kb]
        ).wait()

    @pl.when(nk > 0)
    def _():
        dma_start(0, 0)

    def chunk_body(k, st):
        kb = lax.rem(k, 2)
        dk, b, b16 = bases(k)
        off = b - b16
        dma_wait(k, kb)

        @pl.when(k + 1 < nk)
        def _():
            dma_start(k + 1, 1 - kb)

        def grp_body(q, st):
            cur, regs = st
            lss, xrows, gvs = [], [], []
            for j in range(RU):
                r = RU * q + j
                gi = b + r
                seg = idxp[kb, pl.ds(off + r, L)][0]
                ls = jnp.clip(seg - seg_base, 0, SEGW - 1)
                xrow = [xin[kb, r, pl.ds(L * h, L)] for h in range(HB)]
                paccs = [xrow[p] * wgs[p] for p in range(4)]
                for h in range(4, HB):
                    paccs[h % 4] = paccs[h % 4] + xrow[h] * wgs[h]
                av = ((paccs[0] + paccs[1]) + (paccs[2] + paccs[3])) + bg_init
                z = jnp.sum(av)
                valid = jnp.logical_and(
                    jnp.logical_and(gi >= dk, gi >= lo), gi < hi
                )
                zv = jnp.full((L,), z, jnp.float32)
                gv = (1.0 / (1.0 + jnp.exp(-zv))) * valid.astype(jnp.float32)
                lss.append(ls)
                xrows.append(xrow)
                gvs.append(gv)

            same = (lss[0] == cur) & (lss[1] == cur)

            def fast(cur, regs):
                nregs = []
                for h in range(HB):
                    v = regs[h]
                    for j in range(RU):
                        v = v + xrows[j][h] * gvs[j]
                    nregs.append(v)
                v = regs[HB]
                for j in range(RU):
                    v = v + gvs[j]
                nregs.append(v)
                return cur, tuple(nregs)

            def slow(cur, regs):
                for h in range(HB):
                    acc[cur, pl.ds(L * h, L)] = acc[cur, pl.ds(L * h, L)] + regs[h]
                acc[cur, pl.ds(H, L)] = acc[cur, pl.ds(H, L)] + regs[HB]
                for j in range(RU):
                    for h in range(HB):
                        acc[lss[j], pl.ds(L * h, L)] = (
                            acc[lss[j], pl.ds(L * h, L)] + xrows[j][h] * gvs[j]
                        )
                    acc[lss[j], pl.ds(H, L)] = acc[lss[j], pl.ds(H, L)] + gvs[j]
                zeros = tuple(
                    jnp.zeros((L,), jnp.float32) for _ in range(ZV17)
                )
                return lss[RU - 1], zeros

            return lax.cond(same, fast, slow, cur, regs)

        return lax.fori_loop(0, CH // RU, grp_body, st)

    regs0 = tuple(jnp.zeros((L,), jnp.float32) for _ in range(ZV17))
    cur, regs = lax.fori_loop(0, nk, chunk_body, (jnp.int32(0), regs0))

    for h in range(HB):
        acc[cur, pl.ds(L * h, L)] = acc[cur, pl.ds(L * h, L)] + regs[h]
    acc[cur, pl.ds(H, L)] = acc[cur, pl.ds(H, L)] + regs[HB]

    pltpu.sync_copy(acc, out_hbm.at[pl.ds(seg_base, SEGW), :])


def _sc_segment_accumulate(x, seg, bounds, wgext):
    mesh = plsc.VectorSubcoreMesh(
        core_axis_name="c", subcore_axis_name="s", num_cores=NC, num_subcores=NS
    )
    fn = pl.kernel(
        _sc_body,
        out_type=jax.ShapeDtypeStruct((S, W), jnp.float32),
        mesh=mesh,
        scratch_types=[
            pltpu.VMEM((2, CH, H), jnp.float32),
            pltpu.VMEM((2, CH + 2 * L), jnp.int32),
            pltpu.VMEM((NW + L,), jnp.int32),
            pltpu.VMEM((H + L,), jnp.float32),
            pltpu.VMEM((SEGW, W), jnp.float32),
            pltpu.SemaphoreType.DMA((2,)),
            pltpu.SemaphoreType.DMA((2,)),
        ],
        compiler_params=pltpu.CompilerParams(needs_layout_passes=False),
    )
    return fn(x, seg, bounds, wgext)


def _tc_body(p_ref, wt_ref, bt_ref, o_ref):
    y = p_ref[:, :H]
    g = p_ref[:, H:H + 1]
    o_ref[...] = (
        jnp.dot(y, wt_ref[...], preferred_element_type=jnp.float32) + g * bt_ref[...]
    )


def kernel(node_embedding, segment_ids, Wg, bg, Wt, bt):
    seg = segment_ids.astype(jnp.int32)
    bounds = jnp.searchsorted(
        seg, jnp.arange(0, S + SEGW, SEGW, dtype=jnp.int32)
    ).astype(jnp.int32)
    bounds = jnp.pad(bounds, (0, NW + L - bounds.shape[0]))
    wgext = jnp.concatenate(
        [Wg[:, 0], jnp.full((L,), bg[0] / L, dtype=jnp.float32)]
    )
    partial = _sc_segment_accumulate(node_embedding, seg, bounds, wgext)
    out = pl.pallas_call(
        _tc_body,
        out_shape=jax.ShapeDtypeStruct((S, G), jnp.float32),
    )(partial, Wt, bt.reshape(1, G))
    return out
